# baseline (speedup 1.0000x reference)
import functools

import jax
import jax.numpy as jnp
from jax import lax
from jax.experimental import pallas as pl
from jax.experimental.pallas import tpu as pltpu
from jax.experimental.pallas import tpu_sc as plsc

B, L, F = 4096, 20, 26
VOCAB, DIM = 100000, 32
NC, NS = 2, 16
NW = NC * NS
BPW = B // NW
NB = 2
CH = NB * L * F
GPF = NB * L
NCH = BPW // NB
NG = CH // 16
NPAIR = NCH // 2 - 1


def _build(ncores):
    mesh = plsc.VectorSubcoreMesh(core_axis_name="c", subcore_axis_name="s")

    @functools.partial(
        pl.kernel,
        mesh=mesh,
        out_type=jax.ShapeDtypeStruct((B * L, F, DIM), jnp.float32),
        scratch_types=[
            pltpu.VMEM((NB, L, F), jnp.int32),
            pltpu.VMEM((NB, L, F), jnp.int32),
            pltpu.VMEM((CH,), jnp.int32),
            pltpu.VMEM((CH,), jnp.int32),
            pltpu.VMEM((CH, DIM), jnp.float32),
            pltpu.VMEM((CH, DIM), jnp.float32),
            pltpu.SemaphoreType.DMA,
            pltpu.SemaphoreType.DMA,
            pltpu.SemaphoreType.DMA,
            pltpu.SemaphoreType.DMA,
        ],
        compiler_params=pltpu.CompilerParams(
            use_tc_tiling_on_sc=False, needs_layout_passes=False),
    )
    def run(x_hbm, tab_hbm, out_hbm, raw0, raw1, idx0, idx1, rows0, rows1,
            gsem0, gsem1, osem0, osem1):
        rawb = (raw0, raw1)
        idxb = (idx0, idx1)
        rowsb = (rows0, rows1)
        gsem = (gsem0, gsem1)
        osem = (osem0, osem1)

        wid = lax.axis_index("s") * ncores + lax.axis_index("c")
        b0 = wid * BPW

        def load_transpose(c, b):
            pltpu.sync_copy(x_hbm.at[pl.ds(b0 + c * NB, NB), :, :], rawb[b])

            def grp(m, cy):
                mm = m * 16 + lax.iota(jnp.int32, 16)
                f = mm // GPF
                t = mm % GPF
                v = plsc.load_gather(rawb[b], [t // L, t % L, f])
                idxb[b][pl.ds(m * 16, 16)] = v
                return cy

            lax.fori_loop(0, NG, grp, 0)

        def fire_gathers(b):
            for f in range(F):
                pltpu.async_copy(
                    tab_hbm.at[f].at[idxb[b].at[pl.ds(f * GPF, GPF)]],
                    rowsb[b].at[pl.ds(f * GPF, GPF), :],
                    gsem[b],
                )

        def wait_gathers(b):
            for f in range(F):
                pltpu.make_async_copy(
                    tab_hbm.at[f].at[idxb[b].at[pl.ds(f * GPF, GPF)]],
                    rowsb[b].at[pl.ds(f * GPF, GPF), :],
                    gsem[b],
                ).wait()

        def fire_writebacks(c, b):
            r0 = (b0 + c * NB) * L
            for f in range(F):
                pltpu.async_copy(
                    rowsb[b].at[pl.ds(f * GPF, GPF), :],
                    out_hbm.at[pl.ds(r0, GPF), f, :],
                    osem[b],
                )

        def wait_writebacks(c, b):
            r0 = (b0 + c * NB) * L
            for f in range(F):
                pltpu.make_async_copy(
                    rowsb[b].at[pl.ds(f * GPF, GPF), :],
                    out_hbm.at[pl.ds(r0, GPF), f, :],
                    osem[b],
                ).wait()

        load_transpose(0, 0)
        fire_gathers(0)

        def pair(t, carry):
            c = 2 * t
            wait_gathers(0)
            fire_writebacks(c, 0)
            load_transpose(c + 1, 1)

            @pl.when(t > 0)
            def _():
                wait_writebacks(c - 1, 1)

            fire_gathers(1)
            wait_gathers(1)
            fire_writebacks(c + 1, 1)
            load_transpose(c + 2, 0)
            wait_writebacks(c, 0)
            fire_gathers(0)
            return carry

        lax.fori_loop(0, NPAIR, pair, 0)

        wait_gathers(0)
        fire_writebacks(NCH - 2, 0)
        load_transpose(NCH - 1, 1)
        wait_writebacks(NCH - 3, 1)
        fire_gathers(1)
        wait_gathers(1)
        fire_writebacks(NCH - 1, 1)
        wait_writebacks(NCH - 2, 0)
        wait_writebacks(NCH - 1, 1)

    return run


def kernel(x, tables):
    out = _build(NC)(x.astype(jnp.int32), tables)
    return out.reshape(B, L, F * DIM)


# speedup vs baseline: 1.0914x; 1.0914x over previous
import functools

import jax
import jax.numpy as jnp
from jax import lax
from jax.experimental import pallas as pl
from jax.experimental.pallas import tpu as pltpu
from jax.experimental.pallas import tpu_sc as plsc

B, L, F = 4096, 20, 26
VOCAB, DIM = 100000, 32
NC, NS = 2, 16
NW = NC * NS
NU = F * DIM
UPW = NU // NW
BC = 256
NBC = B // BC
KG = BC // 16
NPAIR = NBC // 2


def _build(ncores):
    mesh = plsc.VectorSubcoreMesh(core_axis_name="c", subcore_axis_name="s")

    @functools.partial(
        pl.kernel,
        mesh=mesh,
        out_type=jax.ShapeDtypeStruct((L, NU, B), jnp.float32),
        scratch_types=[
            pltpu.VMEM((VOCAB,), jnp.float32),
            pltpu.VMEM((L, BC), jnp.int32),
            pltpu.VMEM((L, BC), jnp.int32),
            pltpu.VMEM((L, BC), jnp.float32),
            pltpu.VMEM((L, BC), jnp.float32),
            pltpu.SemaphoreType.DMA,
            pltpu.SemaphoreType.DMA,
        ],
        compiler_params=pltpu.CompilerParams(
            use_tc_tiling_on_sc=False, needs_layout_passes=False),
    )
    def run(xt, tabt, outt, slab, xb0, xb1, ob0, ob1, osem0, osem1):
        xb = (xb0, xb1)
        ob = (ob0, ob1)
        osem = (osem0, osem1)

        wid = lax.axis_index("s") * ncores + lax.axis_index("c")

        def compute(p):
            def lrow(l, cy):
                for k in range(KG):
                    s = pl.ds(k * 16, 16)
                    ob[p][l, s] = plsc.load_gather(slab, [xb[p][l, s]])
                return cy

            lax.fori_loop(0, L, lrow, 0)

        def unit(j, carry):
            u = wid * UPW + j
            f = u // DIM
            d = u % DIM
            pltpu.sync_copy(tabt.at[f, d, :], slab)
            pltpu.sync_copy(xt.at[f, :, pl.ds(0, BC)], xb[0])

            def fire_out(c, p):
                pltpu.async_copy(ob[p], outt.at[:, u, pl.ds(c * BC, BC)],
                                 osem[p])

            def wait_out(c, p):
                pltpu.make_async_copy(ob[p],
                                      outt.at[:, u, pl.ds(c * BC, BC)],
                                      osem[p]).wait()

            def pairs(t, cy):
                c = 2 * t

                @pl.when(t > 0)
                def _():
                    wait_out(c - 2, 0)

                compute(0)
                fire_out(c, 0)
                pltpu.sync_copy(xt.at[f, :, pl.ds((c + 1) * BC, BC)], xb[1])

                @pl.when(t > 0)
                def _():
                    wait_out(c - 1, 1)

                compute(1)
                fire_out(c + 1, 1)

                @pl.when(t < NPAIR - 1)
                def _():
                    pltpu.sync_copy(xt.at[f, :, pl.ds((c + 2) * BC, BC)],
                                    xb[0])

                return cy

            lax.fori_loop(0, NPAIR, pairs, 0)
            wait_out(NBC - 2, 0)
            wait_out(NBC - 1, 1)
            return carry

        lax.fori_loop(0, UPW, unit, 0)

    return run


def kernel(x, tables):
    xt = jnp.transpose(x.astype(jnp.int32), (2, 1, 0))
    tabt = jnp.transpose(tables, (0, 2, 1))
    out_t = _build(NC)(xt, tabt)
    return jnp.transpose(out_t, (2, 0, 1))


# speedup vs baseline: 1.2027x; 1.1021x over previous
import functools

import jax
import jax.numpy as jnp
from jax import lax
from jax.experimental import pallas as pl
from jax.experimental.pallas import tpu as pltpu
from jax.experimental.pallas import tpu_sc as plsc

B, L, F = 4096, 20, 26
VOCAB, DIM = 100000, 32
NC, NS = 2, 16
NW = NC * NS
NU = F * DIM
UPW = NU // NW
BC = 256
NBC = B // BC
KG = BC // 16
NPAIR = NBC // 2


def _build(ncores):
    mesh = plsc.VectorSubcoreMesh(core_axis_name="c", subcore_axis_name="s")

    @functools.partial(
        pl.kernel,
        mesh=mesh,
        out_type=jax.ShapeDtypeStruct((L, NU, B), jnp.float32),
        scratch_types=[
            pltpu.VMEM((VOCAB,), jnp.float32),
            pltpu.VMEM((L, BC), jnp.int32),
            pltpu.VMEM((L, BC), jnp.int32),
            pltpu.VMEM((L, BC), jnp.float32),
            pltpu.VMEM((L, BC), jnp.float32),
            pltpu.SemaphoreType.DMA,
            pltpu.SemaphoreType.DMA,
            pltpu.SemaphoreType.DMA,
            pltpu.SemaphoreType.DMA,
        ],
        compiler_params=pltpu.CompilerParams(
            use_tc_tiling_on_sc=False, needs_layout_passes=False),
    )
    def run(xt, tabt, outt, slab, xb0, xb1, ob0, ob1,
            osem0, osem1, xsem0, xsem1):
        xb = (xb0, xb1)
        ob = (ob0, ob1)
        osem = (osem0, osem1)
        xsem = (xsem0, xsem1)

        wid = lax.axis_index("s") * ncores + lax.axis_index("c")

        def compute(p):
            for l in range(L):
                for k in range(KG):
                    s = pl.ds(k * 16, 16)
                    ob[p][l, s] = plsc.load_gather(slab, [xb[p][l, s]])

        def unit(j, carry):
            u = wid * UPW + j
            f = u // DIM
            d = u % DIM

            def fire_x(c, p):
                pltpu.async_copy(xt.at[f, :, pl.ds(c * BC, BC)], xb[p],
                                 xsem[p])

            def wait_x(c, p):
                pltpu.make_async_copy(xt.at[f, :, pl.ds(c * BC, BC)], xb[p],
                                      xsem[p]).wait()

            def fire_out(c, p):
                pltpu.async_copy(ob[p], outt.at[:, u, pl.ds(c * BC, BC)],
                                 osem[p])

            def wait_out(c, p):
                pltpu.make_async_copy(ob[p],
                                      outt.at[:, u, pl.ds(c * BC, BC)],
                                      osem[p]).wait()

            fire_x(0, 0)
            fire_x(1, 1)
            pltpu.sync_copy(tabt.at[f, d, :], slab)

            def pairs(t, cy):
                c = 2 * t
                wait_x(c, 0)

                @pl.when(t > 0)
                def _():
                    wait_out(c - 2, 0)

                compute(0)
                fire_out(c, 0)

                @pl.when(t < NPAIR - 1)
                def _():
                    fire_x(c + 2, 0)

                wait_x(c + 1, 1)

                @pl.when(t > 0)
                def _():
                    wait_out(c - 1, 1)

                compute(1)
                fire_out(c + 1, 1)

                @pl.when(t < NPAIR - 1)
                def _():
                    fire_x(c + 3, 1)

                return cy

            lax.fori_loop(0, NPAIR, pairs, 0)
            wait_out(NBC - 2, 0)
            wait_out(NBC - 1, 1)
            return carry

        lax.fori_loop(0, UPW, unit, 0)

    return run


def kernel(x, tables):
    xt = jnp.transpose(x.astype(jnp.int32), (2, 1, 0))
    tabt = jnp.transpose(tables, (0, 2, 1))
    out_t = _build(NC)(xt, tabt)
    return jnp.transpose(out_t, (2, 0, 1))


# speedup vs baseline: 1.4028x; 1.1663x over previous
import functools

import jax
import jax.numpy as jnp
from jax import lax
from jax.experimental import pallas as pl
from jax.experimental.pallas import tpu as pltpu
from jax.experimental.pallas import tpu_sc as plsc

B, L, F = 4096, 20, 26
VOCAB, DIM = 100000, 32
NC, NS = 2, 16
NW = NC * NS
NU = F * DIM
UPW = NU // NW
BC = 256
NBC = B // BC
KG = BC // 16
NPAIR = NBC // 2


def _build(ncores):
    mesh = plsc.VectorSubcoreMesh(core_axis_name="c", subcore_axis_name="s")

    @functools.partial(
        pl.kernel,
        mesh=mesh,
        out_type=jax.ShapeDtypeStruct((L, NU // 8, B // 128, 8, 128),
                                      jnp.float32),
        scratch_types=[
            pltpu.VMEM((VOCAB,), jnp.float32),
            pltpu.VMEM((L, BC), jnp.int32),
            pltpu.VMEM((L, BC), jnp.int32),
            pltpu.VMEM((L, BC // 128, 128), jnp.float32),
            pltpu.VMEM((L, BC // 128, 128), jnp.float32),
            pltpu.SemaphoreType.DMA,
            pltpu.SemaphoreType.DMA,
            pltpu.SemaphoreType.DMA,
            pltpu.SemaphoreType.DMA,
        ],
        compiler_params=pltpu.CompilerParams(
            use_tc_tiling_on_sc=False, needs_layout_passes=False),
    )
    def run(xt, tabt, outt, slab, xb0, xb1, ob0, ob1,
            osem0, osem1, xsem0, xsem1):
        xb = (xb0, xb1)
        ob = (ob0, ob1)
        osem = (osem0, osem1)
        xsem = (xsem0, xsem1)

        wid = lax.axis_index("s") * ncores + lax.axis_index("c")

        def compute(p):
            for l in range(L):
                for k in range(KG):
                    s = pl.ds(k * 16, 16)
                    so = pl.ds((k % 8) * 16, 16)
                    ob[p][l, k // 8, so] = plsc.load_gather(slab,
                                                            [xb[p][l, s]])

        def unit(j, carry):
            u = wid * UPW + j
            f = u // DIM
            d = u % DIM

            def fire_x(c, p):
                pltpu.async_copy(xt.at[f, :, pl.ds(c * BC, BC)], xb[p],
                                 xsem[p])

            def wait_x(c, p):
                pltpu.make_async_copy(xt.at[f, :, pl.ds(c * BC, BC)], xb[p],
                                      xsem[p]).wait()

            def fire_out(c, p):
                pltpu.async_copy(
                    ob[p],
                    outt.at[:, u // 8, pl.ds(c * (BC // 128), BC // 128),
                            u % 8, :],
                    osem[p])

            def wait_out(c, p):
                pltpu.make_async_copy(
                    ob[p],
                    outt.at[:, u // 8, pl.ds(c * (BC // 128), BC // 128),
                            u % 8, :],
                    osem[p]).wait()

            fire_x(0, 0)
            fire_x(1, 1)
            pltpu.sync_copy(tabt.at[f, d, :], slab)

            def pairs(t, cy):
                c = 2 * t
                wait_x(c, 0)

                @pl.when(t > 0)
                def _():
                    wait_out(c - 2, 0)

                compute(0)
                fire_out(c, 0)

                @pl.when(t < NPAIR - 1)
                def _():
                    fire_x(c + 2, 0)

                wait_x(c + 1, 1)

                @pl.when(t > 0)
                def _():
                    wait_out(c - 1, 1)

                compute(1)
                fire_out(c + 1, 1)

                @pl.when(t < NPAIR - 1)
                def _():
                    fire_x(c + 3, 1)

                return cy

            lax.fori_loop(0, NPAIR, pairs, 0)
            wait_out(NBC - 2, 0)
            wait_out(NBC - 1, 1)
            return carry

        lax.fori_loop(0, UPW, unit, 0)

    return run


def kernel(x, tables):
    xt = jnp.transpose(x.astype(jnp.int32), (2, 1, 0))
    tabt = jnp.transpose(tables, (0, 2, 1))
    out5 = _build(NC)(xt, tabt)
    return jnp.transpose(out5, (2, 4, 0, 1, 3)).reshape(B, L, F * DIM)


# speedup vs baseline: 1.4052x; 1.0017x over previous
import functools

import jax
import jax.numpy as jnp
from jax import lax
from jax.experimental import pallas as pl
from jax.experimental.pallas import tpu as pltpu
from jax.experimental.pallas import tpu_sc as plsc

B, L, F = 4096, 20, 26
VOCAB, DIM = 100000, 32
NC, NS = 2, 16
NW = NC * NS
NU = F * DIM
UPW = NU // NW
BC = 256
NBC = B // BC
KG = BC // 16
NPAIR = NBC // 2


def _build(ncores):
    mesh = plsc.VectorSubcoreMesh(core_axis_name="c", subcore_axis_name="s")

    @functools.partial(
        pl.kernel,
        mesh=mesh,
        out_type=jax.ShapeDtypeStruct((L, NU // 8, B // 128, 8, 128),
                                      jnp.float32),
        scratch_types=[
            pltpu.VMEM((VOCAB,), jnp.float32),
            pltpu.VMEM((L, BC), jnp.int32),
            pltpu.VMEM((L, BC), jnp.int32),
            pltpu.VMEM((L, BC // 128, 128), jnp.float32),
            pltpu.VMEM((L, BC // 128, 128), jnp.float32),
            pltpu.SemaphoreType.DMA,
            pltpu.SemaphoreType.DMA,
            pltpu.SemaphoreType.DMA,
            pltpu.SemaphoreType.DMA,
            pltpu.SemaphoreType.DMA,
        ],
        compiler_params=pltpu.CompilerParams(
            use_tc_tiling_on_sc=False, needs_layout_passes=False),
    )
    def run(xt, tabt, outt, slab, xb0, xb1, ob0, ob1,
            osem0, osem1, xsem0, xsem1, ssem):
        xb = (xb0, xb1)
        ob = (ob0, ob1)
        osem = (osem0, osem1)
        xsem = (xsem0, xsem1)

        wid = lax.axis_index("s") * ncores + lax.axis_index("c")

        def compute(p):
            for l in range(L):
                for k in range(KG):
                    s = pl.ds(k * 16, 16)
                    so = pl.ds((k % 8) * 16, 16)
                    ob[p][l, k // 8, so] = plsc.load_gather(slab,
                                                            [xb[p][l, s]])

        def unit(j, carry):
            u = wid * UPW + j
            f = u // DIM
            d = u % DIM

            def fire_x(c, p):
                pltpu.async_copy(xt.at[f, :, pl.ds(c * BC, BC)], xb[p],
                                 xsem[p])

            def wait_x(c, p):
                pltpu.make_async_copy(xt.at[f, :, pl.ds(c * BC, BC)], xb[p],
                                      xsem[p]).wait()

            def fire_out(c, p):
                pltpu.async_copy(
                    ob[p],
                    outt.at[:, u // 8, pl.ds(c * (BC // 128), BC // 128),
                            u % 8, :],
                    osem[p])

            def wait_out(c, p):
                pltpu.make_async_copy(
                    ob[p],
                    outt.at[:, u // 8, pl.ds(c * (BC // 128), BC // 128),
                            u % 8, :],
                    osem[p]).wait()

            pltpu.async_copy(tabt.at[f, d, :], slab, ssem)
            fire_x(0, 0)
            fire_x(1, 1)

            @pl.when(j > 0)
            def _():
                up = u - 1
                for (c, p) in ((NBC - 2, 0), (NBC - 1, 1)):
                    pltpu.make_async_copy(
                        ob[p],
                        outt.at[:, up // 8,
                                pl.ds(c * (BC // 128), BC // 128),
                                up % 8, :],
                        osem[p]).wait()

            pltpu.make_async_copy(tabt.at[f, d, :], slab, ssem).wait()

            def pairs(t, cy):
                c = 2 * t
                wait_x(c, 0)

                @pl.when(t > 0)
                def _():
                    wait_out(c - 2, 0)

                compute(0)
                fire_out(c, 0)

                @pl.when(t < NPAIR - 1)
                def _():
                    fire_x(c + 2, 0)

                wait_x(c + 1, 1)

                @pl.when(t > 0)
                def _():
                    wait_out(c - 1, 1)

                compute(1)
                fire_out(c + 1, 1)

                @pl.when(t < NPAIR - 1)
                def _():
                    fire_x(c + 3, 1)

                return cy

            lax.fori_loop(0, NPAIR, pairs, 0)

            @pl.when(j == UPW - 1)
            def _():
                wait_out(NBC - 2, 0)
                wait_out(NBC - 1, 1)

            return carry

        lax.fori_loop(0, UPW, unit, 0)

    return run


def kernel(x, tables):
    xt = jnp.transpose(x.astype(jnp.int32), (2, 1, 0))
    tabt = jnp.transpose(tables, (0, 2, 1))
    out5 = _build(NC)(xt, tabt)
    return jnp.transpose(out5, (2, 4, 0, 1, 3)).reshape(B, L, F * DIM)


# speedup vs baseline: 1.6248x; 1.1563x over previous
import functools

import jax
import jax.numpy as jnp
from jax import lax
from jax.experimental import pallas as pl
from jax.experimental.pallas import tpu as pltpu
from jax.experimental.pallas import tpu_sc as plsc

B, L, F = 4096, 20, 26
VOCAB, DIM = 100000, 32
NC, NS = 2, 16
NW = NC * NS
NU = F * DIM
UPW = NU // NW
BC = 256
NBC = B // BC
KG = BC // 16
NPAIR = NBC // 2


def _build(ncores):
    mesh = plsc.VectorSubcoreMesh(core_axis_name="c", subcore_axis_name="s")

    @functools.partial(
        pl.kernel,
        mesh=mesh,
        out_type=jax.ShapeDtypeStruct((L, NU // 8, B // 128, 8, 128),
                                      jnp.float32),
        scratch_types=[
            pltpu.VMEM((VOCAB,), jnp.float32),
            pltpu.VMEM((L, BC), jnp.int32),
            pltpu.VMEM((L, BC), jnp.int32),
            pltpu.VMEM((L, BC // 128, 128), jnp.float32),
            pltpu.VMEM((L, BC // 128, 128), jnp.float32),
            pltpu.SemaphoreType.DMA,
            pltpu.SemaphoreType.DMA,
            pltpu.SemaphoreType.DMA,
            pltpu.SemaphoreType.DMA,
            pltpu.SemaphoreType.DMA,
        ],
        compiler_params=pltpu.CompilerParams(
            use_tc_tiling_on_sc=False, needs_layout_passes=False),
    )
    def run(xt, tabt, outt, slab, xb0, xb1, ob0, ob1,
            osem0, osem1, xsem0, xsem1, ssem):
        xb = (xb0, xb1)
        ob = (ob0, ob1)
        osem = (osem0, osem1)
        xsem = (xsem0, xsem1)

        wid = lax.axis_index("s") * ncores + lax.axis_index("c")

        def compute(p):
            def lrow(l, cy):
                for k in range(KG):
                    s = pl.ds(k * 16, 16)
                    so = pl.ds((k % 8) * 16, 16)
                    ob[p][l, k // 8, so] = plsc.load_gather(slab,
                                                            [xb[p][l, s]])
                return cy

            lax.fori_loop(0, L, lrow, 0)

        def unit(j, carry):
            u = wid * UPW + j
            f = u // DIM
            d = u % DIM

            def fire_x(c, p):
                pltpu.async_copy(xt.at[f, :, pl.ds(c * BC, BC)], xb[p],
                                 xsem[p])

            def wait_x(c, p):
                pltpu.make_async_copy(xt.at[f, :, pl.ds(c * BC, BC)], xb[p],
                                      xsem[p]).wait()

            def fire_out(c, p):
                pltpu.async_copy(
                    ob[p],
                    outt.at[:, u // 8, pl.ds(c * (BC // 128), BC // 128),
                            u % 8, :],
                    osem[p])

            def wait_out(c, p):
                pltpu.make_async_copy(
                    ob[p],
                    outt.at[:, u // 8, pl.ds(c * (BC // 128), BC // 128),
                            u % 8, :],
                    osem[p]).wait()

            pltpu.async_copy(tabt.at[f, d, :], slab, ssem)
            fire_x(0, 0)
            fire_x(1, 1)

            @pl.when(j > 0)
            def _():
                up = u - 1
                for (c, p) in ((NBC - 2, 0), (NBC - 1, 1)):
                    pltpu.make_async_copy(
                        ob[p],
                        outt.at[:, up // 8,
                                pl.ds(c * (BC // 128), BC // 128),
                                up % 8, :],
                        osem[p]).wait()

            pltpu.make_async_copy(tabt.at[f, d, :], slab, ssem).wait()

            def pairs(t, cy):
                c = 2 * t
                wait_x(c, 0)

                @pl.when(t > 0)
                def _():
                    wait_out(c - 2, 0)

                compute(0)
                fire_out(c, 0)

                @pl.when(t < NPAIR - 1)
                def _():
                    fire_x(c + 2, 0)

                wait_x(c + 1, 1)

                @pl.when(t > 0)
                def _():
                    wait_out(c - 1, 1)

                compute(1)
                fire_out(c + 1, 1)

                @pl.when(t < NPAIR - 1)
                def _():
                    fire_x(c + 3, 1)

                return cy

            lax.fori_loop(0, NPAIR, pairs, 0)

            @pl.when(j == UPW - 1)
            def _():
                wait_out(NBC - 2, 0)
                wait_out(NBC - 1, 1)

            return carry

        lax.fori_loop(0, UPW, unit, 0)

    return run


def kernel(x, tables):
    xt = jnp.transpose(x.astype(jnp.int32), (2, 1, 0))
    tabt = jnp.transpose(tables, (0, 2, 1))
    out5 = _build(NC)(xt, tabt)
    return jnp.transpose(out5, (2, 4, 0, 1, 3)).reshape(B, L, F * DIM)
